# baseline (speedup 1.0000x reference)
import jax
import jax.numpy as jnp
import numpy as np
from jax import lax
from jax.experimental import pallas as pl
from jax.experimental.pallas import tpu as pltpu
from jax.experimental.pallas import tpu_sc as plsc

B, N, K = 4, 8192, 16
C = 128
H = 4
D = 32
HD = H * D
BN = 512
NB = N // BN
SCALE = 1.0 / float(np.sqrt(D))
NTILES = 32
PTS = (B * N) // NTILES
CH = 16
NR = CH * K
W2 = HD // 2


def _wph(Wp):
    head = lax.broadcasted_iota(jnp.int32, (1, HD), 1) // D
    rows = []
    for h in range(H):
        m = (head == h).astype(jnp.float32)
        rows.append(Wp * m)
    return jnp.concatenate(rows, axis=0)


def _bph(bp):
    head = lax.broadcasted_iota(jnp.int32, (1, HD), 1) // D
    rows = []
    for h in range(H):
        m = (head == h).astype(jnp.float32)
        rows.append(bp * m)
    return jnp.concatenate(rows, axis=0)


def _qkv_outputs(f, xyz, Wq, bq, Wk, bk, Wv, bv, Wp, bp):
    ktab = jnp.dot(f, Wk, preferred_element_type=jnp.float32) + bk
    vtab = jnp.dot(f, Wv, preferred_element_type=jnp.float32) + bv
    kv = jnp.concatenate([ktab, vtab], axis=1).astype(jnp.bfloat16)
    q = (jnp.dot(f, Wq, preferred_element_type=jnp.float32) + bq) * SCALE
    WpH = _wph(Wp)
    qp = lax.dot_general(q, WpH, (((1,), (1,)), ((), ())),
                         preferred_element_type=jnp.float32)
    qb = lax.dot_general(q, _bph(bp), (((1,), (1,)), ((), ())),
                         preferred_element_type=jnp.float32)
    aux = jnp.concatenate(
        [xyz, jnp.zeros((BN, 13), jnp.float32), qp, qb,
         jnp.zeros((BN, 96), jnp.float32)], axis=1)
    return kv, q, aux


def _tc1_body(feat_ref, xyz_ref, Win_ref, bin_ref, Wq_ref, bq_ref, Wk_ref,
              bk_ref, Wv_ref, bv_ref, Wp_ref, bp_ref,
              f0_ref, kv_ref, qtab_ref, aux_ref):
    feat = feat_ref[0]
    f = jnp.dot(feat, Win_ref[...], preferred_element_type=jnp.float32) \
        + bin_ref[...]
    f0_ref[0] = f
    kv, q, aux = _qkv_outputs(
        f, xyz_ref[0], Wq_ref[...], bq_ref[...], Wk_ref[...], bk_ref[...],
        Wv_ref[...], bv_ref[...], Wp_ref[...], bp_ref[...])
    kv_ref[...] = kv
    qtab_ref[...] = q
    aux_ref[...] = aux


def _attn_out(ov, wr16, fprev, Wo, bo, Wp, bp):
    attn_v = jnp.dot(ov, Wo, preferred_element_type=jnp.float32)
    WpH = _wph(Wp)
    Mpos = jnp.dot(WpH, Wo, preferred_element_type=jnp.float32)
    posout = jnp.dot(wr16[:, :12], Mpos,
                     preferred_element_type=jnp.float32)
    bprow = jnp.dot(bp, Wo, preferred_element_type=jnp.float32)
    return attn_v + posout + bprow + bo + fprev


def _tc2_body(ov_ref, wr_ref, fprev_ref, xyz_ref, Wo_ref, bo_ref, Wp_ref,
              bp_ref, Wq1_ref, bq1_ref, Wk1_ref, bk1_ref, Wv1_ref, bv1_ref,
              Wp1_ref, bp1_ref,
              f1_ref, kv_ref, qtab_ref, aux_ref):
    f = _attn_out(ov_ref[...], wr_ref[...], fprev_ref[0], Wo_ref[...],
                  bo_ref[...], Wp_ref[...], bp_ref[...])
    f1_ref[0] = f
    kv, q, aux = _qkv_outputs(
        f, xyz_ref[0], Wq1_ref[...], bq1_ref[...], Wk1_ref[...],
        bk1_ref[...], Wv1_ref[...], bv1_ref[...], Wp1_ref[...],
        bp1_ref[...])
    kv_ref[...] = kv
    qtab_ref[...] = q
    aux_ref[...] = aux


def _tc3_body(ov_ref, wr_ref, fprev_ref, Wo_ref, bo_ref, Wp_ref, bp_ref,
              Wout_ref, bout_ref, out_ref):
    f = _attn_out(ov_ref[...], wr_ref[...], fprev_ref[0], Wo_ref[...],
                  bo_ref[...], Wp_ref[...], bp_ref[...])
    out = jnp.dot(f, Wout_ref[...], preferred_element_type=jnp.float32) \
        + bout_ref[...]
    out_ref[0] = jnp.where(out >= 0, out, 0.01 * out)


def _w_spec(shape):
    return pl.BlockSpec(shape, lambda b, n: tuple(0 for _ in shape))


_SPEC_ROWS_C = pl.BlockSpec((1, BN, C), lambda b, n: (b, n, 0))
_SPEC_TAB = pl.BlockSpec((BN, C), lambda b, n: (b * NB + n, 0))
_SPEC_KV = pl.BlockSpec((BN, 2 * HD), lambda b, n: (b * NB + n, 0))
_SPEC_M16 = pl.BlockSpec((BN, 16), lambda b, n: (b * NB + n, 0))


def _tc1(features, xyzs, W_in, b_in, Wq, bq, Wk, bk, Wv, bv, Wp, bp):
    return pl.pallas_call(
        _tc1_body,
        grid=(B, NB),
        in_specs=[
            _SPEC_ROWS_C,
            pl.BlockSpec((1, BN, 3), lambda b, n: (b, n, 0)),
            _w_spec((C, C)), _w_spec((1, C)),
            _w_spec((C, HD)), _w_spec((1, HD)),
            _w_spec((C, HD)), _w_spec((1, HD)),
            _w_spec((C, HD)), _w_spec((1, HD)),
            _w_spec((3, HD)), _w_spec((1, HD)),
        ],
        out_specs=[_SPEC_ROWS_C, _SPEC_KV, _SPEC_TAB, _SPEC_TAB],
        out_shape=[
            jax.ShapeDtypeStruct((B, N, C), jnp.float32),
            jax.ShapeDtypeStruct((B * N, 2 * HD), jnp.bfloat16),
            jax.ShapeDtypeStruct((B * N, C), jnp.float32),
            jax.ShapeDtypeStruct((B * N, C), jnp.float32),
        ],
    )(features, xyzs, W_in, b_in, Wq, bq, Wk, bk, Wv, bv, Wp, bp)


def _tc2(ov, wr, fprev, xyzs, Wo, bo, Wp, bp, Wq1, bq1, Wk1, bk1, Wv1, bv1,
         Wp1, bp1):
    return pl.pallas_call(
        _tc2_body,
        grid=(B, NB),
        in_specs=[
            _SPEC_TAB, _SPEC_M16, _SPEC_ROWS_C,
            pl.BlockSpec((1, BN, 3), lambda b, n: (b, n, 0)),
            _w_spec((HD, C)), _w_spec((1, C)),
            _w_spec((3, HD)), _w_spec((1, HD)),
            _w_spec((C, HD)), _w_spec((1, HD)),
            _w_spec((C, HD)), _w_spec((1, HD)),
            _w_spec((C, HD)), _w_spec((1, HD)),
            _w_spec((3, HD)), _w_spec((1, HD)),
        ],
        out_specs=[_SPEC_ROWS_C, _SPEC_KV, _SPEC_TAB, _SPEC_TAB],
        out_shape=[
            jax.ShapeDtypeStruct((B, N, C), jnp.float32),
            jax.ShapeDtypeStruct((B * N, 2 * HD), jnp.bfloat16),
            jax.ShapeDtypeStruct((B * N, C), jnp.float32),
            jax.ShapeDtypeStruct((B * N, C), jnp.float32),
        ],
    )(ov, wr, fprev, xyzs, Wo, bo, Wp, bp, Wq1, bq1, Wk1, bk1, Wv1, bv1,
      Wp1, bp1)


def _tc3(ov, wr, fprev, Wo, bo, Wp, bp, W_out, b_out):
    return pl.pallas_call(
        _tc3_body,
        grid=(B, NB),
        in_specs=[
            _SPEC_TAB, _SPEC_M16, _SPEC_ROWS_C,
            _w_spec((HD, C)), _w_spec((1, C)),
            _w_spec((3, HD)), _w_spec((1, HD)),
            _w_spec((C, C)), _w_spec((1, C)),
        ],
        out_specs=[_SPEC_ROWS_C],
        out_shape=[jax.ShapeDtypeStruct((B, N, C), jnp.float32)],
    )(ov, wr, fprev, Wo, bo, Wp, bp, W_out, b_out)[0]


def _iota16():
    return lax.iota(jnp.int32, 16)


def _col(ref, c):
    return plsc.load_gather(ref, [_iota16(), jnp.full((16,), c, jnp.int32)])


def _scol(ref, c, val):
    plsc.store_scatter(ref, [_iota16(), jnp.full((16,), c, jnp.int32)], val)


def _pget(ref, r):
    return plsc.load_gather(
        ref, [jnp.full((16,), r // 8, jnp.int32), _iota16() + (r % 8) * 16])


def _pput(ref, r, val):
    plsc.store_scatter(
        ref, [jnp.full((16,), r // 8, jnp.int32), _iota16() + (r % 8) * 16],
        val)


def _unpack2(wvec):
    bf = plsc.bitcast(wvec, jnp.bfloat16)
    return plsc.unpack(bf, format=plsc.PackFormat.INTERLEAVED)


def _sc_body(kv_ref, aux_ref, qtab_ref, xyzst_ref, kg_ref,
             ov_ref, wr_ref,
             idx2, idxg2, kvbuf2, cbuf2, qbuf2,
             xyzvm, rbuf, abuf, obuf, wrbuf, sems):
    wid = lax.axis_index("s") * 2 + lax.axis_index("c")
    base = wid * PTS
    b = base // N
    nloc0 = base - b * N
    boff = b * N
    iota = _iota16()
    NCH = PTS // CH
    pltpu.sync_copy(xyzst_ref.at[b], xyzvm)

    def fire(ci, pr):
        nloc = nloc0 + ci * CH
        gp = base + ci * CH
        po = pr * NR
        poc = pr * CH
        pltpu.sync_copy(kg_ref.at[b, pl.ds(nloc * K, NR)],
                        idx2.at[pl.ds(po, NR)])
        for t in range(CH):
            sl = pl.ds(po + t * 16, 16)
            idxg2[sl] = idx2[sl] + boff
        pltpu.async_copy(kv_ref.at[idxg2.at[pl.ds(po, NR)]],
                         kvbuf2.at[pl.ds(po, NR)], sems.at[pr])
        pltpu.async_copy(aux_ref.at[pl.ds(gp, CH)],
                         cbuf2.at[pl.ds(poc, CH)], sems.at[pr])
        pltpu.async_copy(qtab_ref.at[pl.ds(gp, CH)],
                         qbuf2.at[pl.ds(poc, CH)], sems.at[pr])

    def drain(pr):
        po = pr * NR
        poc = pr * CH
        pltpu.make_async_copy(kv_ref.at[idxg2.at[pl.ds(po, NR)]],
                              kvbuf2.at[pl.ds(po, NR)], sems.at[pr]).wait()
        pltpu.make_async_copy(aux_ref.at[pl.ds(base, CH)],
                              cbuf2.at[pl.ds(poc, CH)], sems.at[pr]).wait()
        pltpu.make_async_copy(qtab_ref.at[pl.ds(base, CH)],
                              qbuf2.at[pl.ds(poc, CH)], sems.at[pr]).wait()

    def compute(ci, pr):
        gp = base + ci * CH
        po = pr * NR
        poc = pr * CH

        def repack(rr):
            rv = jnp.full((16,), po + rr, jnp.int32)
            s = lax.shift_right_logical(rr, 4)
            vals = [plsc.load_gather(kvbuf2, [rv, w0 + iota])
                    for w0 in range(0, HD, 16)]
            for i, w0 in enumerate(range(0, HD, 16)):
                plsc.store_scatter(
                    kvbuf2, [rv, (w0 + s + iota) & (HD - 1)], vals[i])

        plsc.parallel_loop(0, NR, unroll=2)(repack)

        def repackq(rr):
            rv = jnp.full((16,), poc + rr, jnp.int32)
            vals = [plsc.load_gather(qbuf2, [rv, w0 + iota])
                    for w0 in range(0, HD, 16)]
            for i, w0 in enumerate(range(0, HD, 16)):
                plsc.store_scatter(
                    qbuf2, [rv, (w0 + rr + iota) & (HD - 1)], vals[i])

        plsc.parallel_loop(0, CH)(repackq)
        rowvecs = []
        for j in range(K):
            raw = plsc.load_gather(idx2, [iota * K + j + po])
            rowvecs.append(iota * K + j + po)
            xr = lax.shift_right_logical(raw, 6)
            xc = (raw & 63) * 2
            nx, ny = _unpack2(plsc.load_gather(xyzvm, [xr, xc]))
            nz, _ = _unpack2(plsc.load_gather(xyzvm, [xr, xc + 1]))
            _pput(rbuf, 0 * K + j, nx)
            _pput(rbuf, 1 * K + j, ny)
            _pput(rbuf, 2 * K + j, nz)

        def _cc(c):
            return plsc.load_gather(
                cbuf2, [iota + poc, jnp.full((16,), c, jnp.int32)])

        for c in range(3):
            cvec = _cc(c)
            for j in range(K):
                _pput(rbuf, c * K + j, _pget(rbuf, c * K + j) - cvec)
        for h in range(H):
            qp_c = [_cc(16 + h * 3 + c) for c in range(3)]
            qb_h = _cc(28 + h)
            logit = []
            for j in range(K):
                lj = qb_h
                for c in range(3):
                    lj = lj + qp_c[c] * _pget(rbuf, c * K + j)
                logit.append(lj)

            def dd_body(w, carry):
                wg = h * (D // 2) + w
                qv0 = plsc.load_gather(
                    qbuf2, [iota + poc, (2 * wg + iota) & (HD - 1)])
                qv1 = plsc.load_gather(
                    qbuf2, [iota + poc, (2 * wg + 1 + iota) & (HD - 1)])
                wgv = (wg + iota) & (HD - 1)
                out = []
                for j in range(K):
                    k0, k1 = _unpack2(
                        plsc.load_gather(kvbuf2, [rowvecs[j], wgv]))
                    out.append(carry[j] + qv0 * k0 + qv1 * k1)
                return tuple(out)

            logit = plsc.parallel_loop(0, D // 2, unroll=16,
                                       carry=tuple(logit))(dd_body)
            m = logit[0]
            for j in range(1, K):
                m = jnp.maximum(m, logit[j])
            es = [jnp.exp(logit[j] - m) for j in range(K)]
            s = es[0]
            for j in range(1, K):
                s = s + es[j]
            rinv = 1.0 / s
            attn = [es[j] * rinv for j in range(K)]
            for j in range(K):
                _pput(abuf, h * K + j, attn[j])
            for c in range(3):
                acc = attn[0] * _pget(rbuf, c * K + 0)
                for j in range(1, K):
                    acc = acc + attn[j] * _pget(rbuf, c * K + j)
                _scol(wrbuf, h * 3 + c, acc)
        for h in range(H):
            a = [_pget(abuf, h * K + j) for j in range(K)]

            def vv_body(w):
                wg = W2 + h * (D // 2) + w
                wgv = (wg + iota) & (HD - 1)
                v0, v1 = _unpack2(
                    plsc.load_gather(kvbuf2, [rowvecs[0], wgv]))
                acc0 = a[0] * v0
                acc1 = a[0] * v1
                for j in range(1, K):
                    v0, v1 = _unpack2(
                        plsc.load_gather(kvbuf2, [rowvecs[j], wgv]))
                    acc0 = acc0 + a[j] * v0
                    acc1 = acc1 + a[j] * v1
                col = h * D + 2 * w
                plsc.store_scatter(
                    obuf, [iota, (col + iota) & (HD - 1)], acc0)
                plsc.store_scatter(
                    obuf, [iota, (col + 1 + iota) & (HD - 1)], acc1)

            plsc.parallel_loop(0, D // 2, unroll=16)(vv_body)

        def unsk(rr):
            rv = jnp.full((16,), rr, jnp.int32)
            vals = [plsc.load_gather(
                        obuf, [rv, (w0 + rr + iota) & (HD - 1)])
                    for w0 in range(0, HD, 16)]
            for i, w0 in enumerate(range(0, HD, 16)):
                plsc.store_scatter(obuf, [rv, w0 + iota], vals[i])

        plsc.parallel_loop(0, CH)(unsk)
        pltpu.sync_copy(obuf, ov_ref.at[pl.ds(gp, CH)])
        pltpu.sync_copy(wrbuf, wr_ref.at[pl.ds(gp, CH)])

    fire(0, jnp.int32(0))

    def body(ci, _):
        pr = ci & 1
        cn = jnp.minimum(ci + 1, NCH - 1)
        fire(cn, 1 - pr)
        drain(pr)
        compute(ci, pr)
        return 0

    lax.fori_loop(0, NCH, body, 0)
    drain(jnp.int32(NCH & 1))


def _sc_attention(kv32, aux, qtab, xyzst, kgflat):
    mesh = plsc.VectorSubcoreMesh(core_axis_name="c", subcore_axis_name="s")
    fn = pl.kernel(
        _sc_body,
        out_type=[
            jax.ShapeDtypeStruct((B * N, HD), jnp.float32),
            jax.ShapeDtypeStruct((B * N, 16), jnp.float32),
        ],
        mesh=mesh,
        compiler_params=pltpu.CompilerParams(needs_layout_passes=False),
        scratch_types=[
            pltpu.VMEM((2 * NR,), jnp.int32),
            pltpu.VMEM((2 * NR,), jnp.int32),
            pltpu.VMEM((2 * NR, HD), jnp.int32),
            pltpu.VMEM((2 * CH, HD), jnp.float32),
            pltpu.VMEM((2 * CH, HD), jnp.float32),
            pltpu.VMEM((N // 64, HD), jnp.int32),
            pltpu.VMEM((6, HD), jnp.float32),
            pltpu.VMEM((8, HD), jnp.float32),
            pltpu.VMEM((CH, HD), jnp.float32),
            pltpu.VMEM((CH, 16), jnp.float32),
            pltpu.SemaphoreType.DMA((2,)),
        ],
    )
    return fn(kv32, aux, qtab, xyzst, kgflat)


def kernel(xyzs, features, k_graph, W_in, b_in, Wq0, bq0, Wk0, bk0, Wv0, bv0,
           Wp0, bp0, Wo0, bo0, Wq1, bq1, Wk1, bk1, Wv1, bv1, Wp1, bp1, Wo1,
           bo1, W_out, b_out):
    r2 = lambda v: v.reshape(1, -1)
    kgflat = k_graph.reshape(B, N * K)
    xyzst = lax.bitcast_convert_type(
        jnp.concatenate(
            [xyzs, jnp.zeros((B, N, 1), jnp.float32)], axis=-1
        ).astype(jnp.bfloat16).reshape(B, N * 2, 2),
        jnp.int32).reshape(B, N // 64, HD)
    as32 = lambda kv: lax.bitcast_convert_type(
        kv.reshape(B * N, HD, 2), jnp.int32)
    f0, kv0, q0, aux0 = _tc1(
        features, xyzs, W_in, r2(b_in), Wq0, r2(bq0), Wk0, r2(bk0),
        Wv0, r2(bv0), Wp0, r2(bp0))
    ov0, wr0 = _sc_attention(as32(kv0), aux0, q0, xyzst, kgflat)
    f1, kv1, q1, aux1 = _tc2(
        ov0, wr0, f0, xyzs, Wo0, r2(bo0), Wp0, r2(bp0), Wq1, r2(bq1), Wk1,
        r2(bk1), Wv1, r2(bv1), Wp1, r2(bp1))
    ov1, wr1 = _sc_attention(as32(kv1), aux1, q1, xyzst, kgflat)
    return _tc3(ov1, wr1, f1, Wo1, r2(bo1), Wp1, r2(bp1), W_out, r2(b_out))


# speedup vs baseline: 1.2631x; 1.2631x over previous
import jax
import jax.numpy as jnp
import numpy as np
from jax import lax
from jax.experimental import pallas as pl
from jax.experimental.pallas import tpu as pltpu
from jax.experimental.pallas import tpu_sc as plsc

B, N, K = 4, 8192, 16
C = 128
H = 4
D = 32
HD = H * D
BN = 512
NB = N // BN
SCALE = 1.0 / float(np.sqrt(D))
NTILES = 32
PTS = (B * N) // NTILES
CH = 16
NR = CH * K
W2 = HD // 2


def _wph(Wp):
    head = lax.broadcasted_iota(jnp.int32, (1, HD), 1) // D
    rows = []
    for h in range(H):
        m = (head == h).astype(jnp.float32)
        rows.append(Wp * m)
    return jnp.concatenate(rows, axis=0)


def _bph(bp):
    head = lax.broadcasted_iota(jnp.int32, (1, HD), 1) // D
    rows = []
    for h in range(H):
        m = (head == h).astype(jnp.float32)
        rows.append(bp * m)
    return jnp.concatenate(rows, axis=0)


def _qkv_outputs(f, xyz, Wq, bq, Wk, bk, Wv, bv, Wp, bp):
    ktab = jnp.dot(f, Wk, preferred_element_type=jnp.float32) + bk
    vtab = jnp.dot(f, Wv, preferred_element_type=jnp.float32) + bv
    kv = jnp.concatenate([ktab, vtab], axis=1).astype(jnp.bfloat16)
    q = (jnp.dot(f, Wq, preferred_element_type=jnp.float32) + bq) * SCALE
    WpH = _wph(Wp)
    qp = lax.dot_general(q, WpH, (((1,), (1,)), ((), ())),
                         preferred_element_type=jnp.float32)
    qb = lax.dot_general(q, _bph(bp), (((1,), (1,)), ((), ())),
                         preferred_element_type=jnp.float32)
    aux = jnp.concatenate(
        [xyz, jnp.zeros((BN, 13), jnp.float32), qp, qb,
         jnp.zeros((BN, 96), jnp.float32)], axis=1)
    return kv, q, aux


def _tc1_body(feat_ref, xyz_ref, Win_ref, bin_ref, Wq_ref, bq_ref, Wk_ref,
              bk_ref, Wv_ref, bv_ref, Wp_ref, bp_ref,
              f0_ref, kv_ref, qtab_ref, aux_ref):
    feat = feat_ref[0]
    f = jnp.dot(feat, Win_ref[...], preferred_element_type=jnp.float32) \
        + bin_ref[...]
    f0_ref[0] = f
    kv, q, aux = _qkv_outputs(
        f, xyz_ref[0], Wq_ref[...], bq_ref[...], Wk_ref[...], bk_ref[...],
        Wv_ref[...], bv_ref[...], Wp_ref[...], bp_ref[...])
    kv_ref[...] = kv
    qtab_ref[...] = q
    aux_ref[...] = aux


def _attn_out(ov, wr16, fprev, Wo, bo, Wp, bp):
    attn_v = jnp.dot(ov, Wo, preferred_element_type=jnp.float32)
    WpH = _wph(Wp)
    Mpos = jnp.dot(WpH, Wo, preferred_element_type=jnp.float32)
    posout = jnp.dot(wr16[:, :12], Mpos,
                     preferred_element_type=jnp.float32)
    bprow = jnp.dot(bp, Wo, preferred_element_type=jnp.float32)
    return attn_v + posout + bprow + bo + fprev


def _tc2_body(ov_ref, wr_ref, fprev_ref, xyz_ref, Wo_ref, bo_ref, Wp_ref,
              bp_ref, Wq1_ref, bq1_ref, Wk1_ref, bk1_ref, Wv1_ref, bv1_ref,
              Wp1_ref, bp1_ref,
              f1_ref, kv_ref, qtab_ref, aux_ref):
    f = _attn_out(ov_ref[...], wr_ref[...], fprev_ref[0], Wo_ref[...],
                  bo_ref[...], Wp_ref[...], bp_ref[...])
    f1_ref[0] = f
    kv, q, aux = _qkv_outputs(
        f, xyz_ref[0], Wq1_ref[...], bq1_ref[...], Wk1_ref[...],
        bk1_ref[...], Wv1_ref[...], bv1_ref[...], Wp1_ref[...],
        bp1_ref[...])
    kv_ref[...] = kv
    qtab_ref[...] = q
    aux_ref[...] = aux


def _tc3_body(ov_ref, wr_ref, fprev_ref, Wo_ref, bo_ref, Wp_ref, bp_ref,
              Wout_ref, bout_ref, out_ref):
    f = _attn_out(ov_ref[...], wr_ref[...], fprev_ref[0], Wo_ref[...],
                  bo_ref[...], Wp_ref[...], bp_ref[...])
    out = jnp.dot(f, Wout_ref[...], preferred_element_type=jnp.float32) \
        + bout_ref[...]
    out_ref[0] = jnp.where(out >= 0, out, 0.01 * out)


def _w_spec(shape):
    return pl.BlockSpec(shape, lambda b, n: tuple(0 for _ in shape))


_SPEC_ROWS_C = pl.BlockSpec((1, BN, C), lambda b, n: (b, n, 0))
_SPEC_TAB = pl.BlockSpec((BN, C), lambda b, n: (b * NB + n, 0))
_SPEC_KV = pl.BlockSpec((BN, 2 * HD), lambda b, n: (b * NB + n, 0))
_SPEC_M16 = pl.BlockSpec((BN, 16), lambda b, n: (b * NB + n, 0))


def _tc1(features, xyzs, W_in, b_in, Wq, bq, Wk, bk, Wv, bv, Wp, bp):
    return pl.pallas_call(
        _tc1_body,
        grid=(B, NB),
        in_specs=[
            _SPEC_ROWS_C,
            pl.BlockSpec((1, BN, 3), lambda b, n: (b, n, 0)),
            _w_spec((C, C)), _w_spec((1, C)),
            _w_spec((C, HD)), _w_spec((1, HD)),
            _w_spec((C, HD)), _w_spec((1, HD)),
            _w_spec((C, HD)), _w_spec((1, HD)),
            _w_spec((3, HD)), _w_spec((1, HD)),
        ],
        out_specs=[_SPEC_ROWS_C, _SPEC_KV, _SPEC_TAB, _SPEC_TAB],
        out_shape=[
            jax.ShapeDtypeStruct((B, N, C), jnp.float32),
            jax.ShapeDtypeStruct((B * N, 2 * HD), jnp.bfloat16),
            jax.ShapeDtypeStruct((B * N, C), jnp.float32),
            jax.ShapeDtypeStruct((B * N, C), jnp.float32),
        ],
    )(features, xyzs, W_in, b_in, Wq, bq, Wk, bk, Wv, bv, Wp, bp)


def _tc2(ov, wr, fprev, xyzs, Wo, bo, Wp, bp, Wq1, bq1, Wk1, bk1, Wv1, bv1,
         Wp1, bp1):
    return pl.pallas_call(
        _tc2_body,
        grid=(B, NB),
        in_specs=[
            _SPEC_TAB, _SPEC_M16, _SPEC_ROWS_C,
            pl.BlockSpec((1, BN, 3), lambda b, n: (b, n, 0)),
            _w_spec((HD, C)), _w_spec((1, C)),
            _w_spec((3, HD)), _w_spec((1, HD)),
            _w_spec((C, HD)), _w_spec((1, HD)),
            _w_spec((C, HD)), _w_spec((1, HD)),
            _w_spec((C, HD)), _w_spec((1, HD)),
            _w_spec((3, HD)), _w_spec((1, HD)),
        ],
        out_specs=[_SPEC_ROWS_C, _SPEC_KV, _SPEC_TAB, _SPEC_TAB],
        out_shape=[
            jax.ShapeDtypeStruct((B, N, C), jnp.float32),
            jax.ShapeDtypeStruct((B * N, 2 * HD), jnp.bfloat16),
            jax.ShapeDtypeStruct((B * N, C), jnp.float32),
            jax.ShapeDtypeStruct((B * N, C), jnp.float32),
        ],
    )(ov, wr, fprev, xyzs, Wo, bo, Wp, bp, Wq1, bq1, Wk1, bk1, Wv1, bv1,
      Wp1, bp1)


def _tc3(ov, wr, fprev, Wo, bo, Wp, bp, W_out, b_out):
    return pl.pallas_call(
        _tc3_body,
        grid=(B, NB),
        in_specs=[
            _SPEC_TAB, _SPEC_M16, _SPEC_ROWS_C,
            _w_spec((HD, C)), _w_spec((1, C)),
            _w_spec((3, HD)), _w_spec((1, HD)),
            _w_spec((C, C)), _w_spec((1, C)),
        ],
        out_specs=[_SPEC_ROWS_C],
        out_shape=[jax.ShapeDtypeStruct((B, N, C), jnp.float32)],
    )(ov, wr, fprev, Wo, bo, Wp, bp, W_out, b_out)[0]


def _iota16():
    return lax.iota(jnp.int32, 16)


def _col(ref, c):
    return plsc.load_gather(ref, [_iota16(), jnp.full((16,), c, jnp.int32)])


def _scol(ref, c, val):
    plsc.store_scatter(ref, [_iota16(), jnp.full((16,), c, jnp.int32)], val)


def _pget(ref, r):
    return plsc.load_gather(
        ref, [jnp.full((16,), r // 8, jnp.int32), _iota16() + (r % 8) * 16])


def _pput(ref, r, val):
    plsc.store_scatter(
        ref, [jnp.full((16,), r // 8, jnp.int32), _iota16() + (r % 8) * 16],
        val)


def _unpack2(wvec):
    bf = plsc.bitcast(wvec, jnp.bfloat16)
    return plsc.unpack(bf, format=plsc.PackFormat.INTERLEAVED)


def _sc_body(kv_ref, aux_ref, qtab_ref, xyzst_ref, kg_ref,
             ov_ref, wr_ref,
             idx2, idxg2, kvbuf2, cbuf2, qbuf2,
             xyzvm, rbuf, abuf, obuf, wrbuf, sems, semo):
    wid = lax.axis_index("s") * 2 + lax.axis_index("c")
    base = wid * PTS
    b = base // N
    nloc0 = base - b * N
    boff = b * N
    iota = _iota16()
    NCH = PTS // CH
    pltpu.sync_copy(xyzst_ref.at[b], xyzvm)

    def owait(pr):
        poc = pr * CH
        pltpu.make_async_copy(obuf.at[pl.ds(poc, CH)],
                              ov_ref.at[pl.ds(base, CH)],
                              semo.at[pr]).wait()
        pltpu.make_async_copy(wrbuf.at[pl.ds(poc, CH)],
                              wr_ref.at[pl.ds(base, CH)],
                              semo.at[pr]).wait()

    for p in range(2):
        pltpu.async_copy(ov_ref.at[pl.ds(base, CH)],
                         obuf.at[pl.ds(p * CH, CH)], semo.at[p])
        pltpu.async_copy(wr_ref.at[pl.ds(base, CH)],
                         wrbuf.at[pl.ds(p * CH, CH)], semo.at[p])

    def fire(ci, pr):
        nloc = nloc0 + ci * CH
        gp = base + ci * CH
        po = pr * NR
        poc = pr * CH
        pltpu.sync_copy(kg_ref.at[b, pl.ds(nloc * K, NR)],
                        idx2.at[pl.ds(po, NR)])
        for t in range(CH):
            sl = pl.ds(po + t * 16, 16)
            idxg2[sl] = idx2[sl] + boff
        pltpu.async_copy(kv_ref.at[idxg2.at[pl.ds(po, NR)]],
                         kvbuf2.at[pl.ds(po, NR)], sems.at[pr])
        pltpu.async_copy(aux_ref.at[pl.ds(gp, CH)],
                         cbuf2.at[pl.ds(poc, CH)], sems.at[pr])
        pltpu.async_copy(qtab_ref.at[pl.ds(gp, CH)],
                         qbuf2.at[pl.ds(poc, CH)], sems.at[pr])

    def drain(pr):
        po = pr * NR
        poc = pr * CH
        pltpu.make_async_copy(kv_ref.at[idxg2.at[pl.ds(po, NR)]],
                              kvbuf2.at[pl.ds(po, NR)], sems.at[pr]).wait()
        pltpu.make_async_copy(aux_ref.at[pl.ds(base, CH)],
                              cbuf2.at[pl.ds(poc, CH)], sems.at[pr]).wait()
        pltpu.make_async_copy(qtab_ref.at[pl.ds(base, CH)],
                              qbuf2.at[pl.ds(poc, CH)], sems.at[pr]).wait()

    def compute(ci, pr):
        gp = base + ci * CH
        po = pr * NR
        poc = pr * CH
        owait(pr)

        def repack(rr):
            rv = jnp.full((16,), po + rr, jnp.int32)
            s = lax.shift_right_logical(rr, 4)
            vals = [plsc.load_gather(kvbuf2, [rv, w0 + iota])
                    for w0 in range(0, HD, 16)]
            for i, w0 in enumerate(range(0, HD, 16)):
                plsc.store_scatter(
                    kvbuf2, [rv, (w0 + s + iota) & (HD - 1)], vals[i])

        plsc.parallel_loop(0, NR, unroll=2)(repack)

        def repackq(rr):
            rv = jnp.full((16,), poc + rr, jnp.int32)
            vals = [plsc.load_gather(qbuf2, [rv, w0 + iota])
                    for w0 in range(0, HD, 16)]
            for i, w0 in enumerate(range(0, HD, 16)):
                plsc.store_scatter(
                    qbuf2, [rv, (w0 + rr + iota) & (HD - 1)], vals[i])

        plsc.parallel_loop(0, CH)(repackq)
        rowvecs = []
        for j in range(K):
            raw = plsc.load_gather(idx2, [iota * K + j + po])
            rowvecs.append(iota * K + j + po)
            xr = lax.shift_right_logical(raw, 6)
            xc = (raw & 63) * 2
            nx, ny = _unpack2(plsc.load_gather(xyzvm, [xr, xc]))
            nz, _ = _unpack2(plsc.load_gather(xyzvm, [xr, xc + 1]))
            _pput(rbuf, 0 * K + j, nx)
            _pput(rbuf, 1 * K + j, ny)
            _pput(rbuf, 2 * K + j, nz)

        def _cc(c):
            return plsc.load_gather(
                cbuf2, [iota + poc, jnp.full((16,), c, jnp.int32)])

        for c in range(3):
            cvec = _cc(c)
            for j in range(K):
                _pput(rbuf, c * K + j, _pget(rbuf, c * K + j) - cvec)
        for h in range(H):
            qp_c = [_cc(16 + h * 3 + c) for c in range(3)]
            qb_h = _cc(28 + h)
            logit = []
            for j in range(K):
                lj = qb_h
                for c in range(3):
                    lj = lj + qp_c[c] * _pget(rbuf, c * K + j)
                logit.append(lj)

            def dd_body(w, carry):
                wg = h * (D // 2) + w
                qv0 = plsc.load_gather(
                    qbuf2, [iota + poc, (2 * wg + iota) & (HD - 1)])
                qv1 = plsc.load_gather(
                    qbuf2, [iota + poc, (2 * wg + 1 + iota) & (HD - 1)])
                wgv = (wg + iota) & (HD - 1)
                out = []
                for j in range(K):
                    k0, k1 = _unpack2(
                        plsc.load_gather(kvbuf2, [rowvecs[j], wgv]))
                    out.append(carry[j] + qv0 * k0 + qv1 * k1)
                return tuple(out)

            logit = plsc.parallel_loop(0, D // 2, unroll=8,
                                       carry=tuple(logit))(dd_body)
            m = logit[0]
            for j in range(1, K):
                m = jnp.maximum(m, logit[j])
            es = [jnp.exp(logit[j] - m) for j in range(K)]
            s = es[0]
            for j in range(1, K):
                s = s + es[j]
            rinv = 1.0 / s
            attn = [es[j] * rinv for j in range(K)]
            for j in range(K):
                _pput(abuf, h * K + j, attn[j])
            for c in range(3):
                acc = attn[0] * _pget(rbuf, c * K + 0)
                for j in range(1, K):
                    acc = acc + attn[j] * _pget(rbuf, c * K + j)
                plsc.store_scatter(
                    wrbuf,
                    [iota + poc, jnp.full((16,), h * 3 + c, jnp.int32)], acc)
        for h in range(H):
            a = [_pget(abuf, h * K + j) for j in range(K)]

            def vv_body(w):
                wg = W2 + h * (D // 2) + w
                wgv = (wg + iota) & (HD - 1)
                v0, v1 = _unpack2(
                    plsc.load_gather(kvbuf2, [rowvecs[0], wgv]))
                acc0 = a[0] * v0
                acc1 = a[0] * v1
                for j in range(1, K):
                    v0, v1 = _unpack2(
                        plsc.load_gather(kvbuf2, [rowvecs[j], wgv]))
                    acc0 = acc0 + a[j] * v0
                    acc1 = acc1 + a[j] * v1
                col = h * D + 2 * w
                plsc.store_scatter(
                    obuf, [iota + poc, (col + iota) & (HD - 1)], acc0)
                plsc.store_scatter(
                    obuf, [iota + poc, (col + 1 + iota) & (HD - 1)], acc1)

            plsc.parallel_loop(0, D // 2, unroll=8)(vv_body)

        def unsk(rr):
            rv = jnp.full((16,), poc + rr, jnp.int32)
            vals = [plsc.load_gather(
                        obuf, [rv, (w0 + rr + iota) & (HD - 1)])
                    for w0 in range(0, HD, 16)]
            for i, w0 in enumerate(range(0, HD, 16)):
                plsc.store_scatter(obuf, [rv, w0 + iota], vals[i])

        plsc.parallel_loop(0, CH)(unsk)
        pltpu.async_copy(obuf.at[pl.ds(poc, CH)],
                         ov_ref.at[pl.ds(gp, CH)], semo.at[pr])
        pltpu.async_copy(wrbuf.at[pl.ds(poc, CH)],
                         wr_ref.at[pl.ds(gp, CH)], semo.at[pr])

    fire(0, jnp.int32(0))

    def body(ci, _):
        pr = ci & 1
        cn = jnp.minimum(ci + 1, NCH - 1)
        fire(cn, 1 - pr)
        drain(pr)
        compute(ci, pr)
        return 0

    lax.fori_loop(0, NCH, body, 0)
    drain(jnp.int32(NCH & 1))
    owait(jnp.int32(0))
    owait(jnp.int32(1))


def _sc_attention(kv32, aux, qtab, xyzst, kgflat):
    mesh = plsc.VectorSubcoreMesh(core_axis_name="c", subcore_axis_name="s")
    fn = pl.kernel(
        _sc_body,
        out_type=[
            jax.ShapeDtypeStruct((B * N, HD), jnp.float32),
            jax.ShapeDtypeStruct((B * N, 16), jnp.float32),
        ],
        mesh=mesh,
        compiler_params=pltpu.CompilerParams(needs_layout_passes=False),
        scratch_types=[
            pltpu.VMEM((2 * NR,), jnp.int32),
            pltpu.VMEM((2 * NR,), jnp.int32),
            pltpu.VMEM((2 * NR, HD), jnp.int32),
            pltpu.VMEM((2 * CH, HD), jnp.float32),
            pltpu.VMEM((2 * CH, HD), jnp.float32),
            pltpu.VMEM((N // 64, HD), jnp.int32),
            pltpu.VMEM((6, HD), jnp.float32),
            pltpu.VMEM((8, HD), jnp.float32),
            pltpu.VMEM((2 * CH, HD), jnp.float32),
            pltpu.VMEM((2 * CH, 16), jnp.float32),
            pltpu.SemaphoreType.DMA((2,)),
            pltpu.SemaphoreType.DMA((2,)),
        ],
    )
    return fn(kv32, aux, qtab, xyzst, kgflat)


def kernel(xyzs, features, k_graph, W_in, b_in, Wq0, bq0, Wk0, bk0, Wv0, bv0,
           Wp0, bp0, Wo0, bo0, Wq1, bq1, Wk1, bk1, Wv1, bv1, Wp1, bp1, Wo1,
           bo1, W_out, b_out):
    r2 = lambda v: v.reshape(1, -1)
    kgflat = k_graph.reshape(B, N * K)
    xyzst = lax.bitcast_convert_type(
        jnp.concatenate(
            [xyzs, jnp.zeros((B, N, 1), jnp.float32)], axis=-1
        ).astype(jnp.bfloat16).reshape(B, N * 2, 2),
        jnp.int32).reshape(B, N // 64, HD)
    as32 = lambda kv: lax.bitcast_convert_type(
        kv.reshape(B * N, HD, 2), jnp.int32)
    f0, kv0, q0, aux0 = _tc1(
        features, xyzs, W_in, r2(b_in), Wq0, r2(bq0), Wk0, r2(bk0),
        Wv0, r2(bv0), Wp0, r2(bp0))
    ov0, wr0 = _sc_attention(as32(kv0), aux0, q0, xyzst, kgflat)
    f1, kv1, q1, aux1 = _tc2(
        ov0, wr0, f0, xyzs, Wo0, r2(bo0), Wp0, r2(bp0), Wq1, r2(bq1), Wk1,
        r2(bk1), Wv1, r2(bv1), Wp1, r2(bp1))
    ov1, wr1 = _sc_attention(as32(kv1), aux1, q1, xyzst, kgflat)
    return _tc3(ov1, wr1, f1, Wo1, r2(bo1), Wp1, r2(bp1), W_out, r2(b_out))
